# Initial kernel scaffold; baseline (speedup 1.0000x reference)
import functools

import jax
import jax.numpy as jnp
from jax import lax
from jax.experimental import pallas as pl
from jax.experimental.pallas import tpu as pltpu
from jax.experimental.pallas import tpu_sc as plsc

N_POINTS = 16384
N_FEAT = 64
N_CLUSTERS = 128
BLK = 2048
N_BLK = N_POINTS // BLK


def _argmin_body(x_ref, c_ref, idx_ref, bv_ref, bi_ref):
    j = pl.program_id(0)
    xb = x_ref[...]
    cb = c_ref[...]
    dot = lax.dot_general(
        xb, cb, (((1,), (1,)), ((), ())), preferred_element_type=jnp.float32
    )
    xn = jnp.sum(xb * xb, axis=1, keepdims=True)
    s = xn - 2.0 * dot
    bmin = jnp.min(s, axis=0, keepdims=True)
    ii = lax.broadcasted_iota(jnp.int32, s.shape, 0) + j * BLK
    bidx = jnp.min(
        jnp.where(s == bmin, ii, jnp.int32(2**30)), axis=0, keepdims=True
    )

    @pl.when(j == 0)
    def _():
        bv_ref[...] = bmin
        bi_ref[...] = bidx

    @pl.when(j > 0)
    def _():
        better = bmin < bv_ref[...]
        bv_ref[...] = jnp.where(better, bmin, bv_ref[...])
        bi_ref[...] = jnp.where(better, bidx, bi_ref[...])

    @pl.when(j == N_BLK - 1)
    def _():
        idx_ref[...] = bi_ref[...]


def _tc_argmin(x2d, centers):
    return pl.pallas_call(
        _argmin_body,
        grid=(N_BLK,),
        in_specs=[
            pl.BlockSpec((BLK, N_FEAT), lambda j: (j, 0)),
            pl.BlockSpec((N_CLUSTERS, N_FEAT), lambda j: (0, 0)),
        ],
        out_specs=pl.BlockSpec((1, N_CLUSTERS), lambda j: (0, 0)),
        out_shape=jax.ShapeDtypeStruct((1, N_CLUSTERS), jnp.int32),
        scratch_shapes=[
            pltpu.VMEM((1, N_CLUSTERS), jnp.float32),
            pltpu.VMEM((1, N_CLUSTERS), jnp.int32),
        ],
    )(x2d, centers)


_SC_INFO = plsc.get_sparse_core_info()
_NC = _SC_INFO.num_cores
_GATHER_WORKERS = 16
_ROWS_PER_WORKER = N_CLUSTERS // _GATHER_WORKERS


@functools.partial(
    pl.kernel,
    mesh=plsc.VectorSubcoreMesh(core_axis_name="c", subcore_axis_name="s"),
    out_type=jax.ShapeDtypeStruct((N_CLUSTERS, N_FEAT), jnp.float32),
    scratch_types=[
        pltpu.VMEM((_ROWS_PER_WORKER,), jnp.int32),
        pltpu.VMEM((_ROWS_PER_WORKER, N_FEAT), jnp.float32),
        pltpu.SemaphoreType.DMA,
    ],
)
def _sc_gather(x_hbm, idx_hbm, out_hbm, idx_v, rows_v, sem):
    wid = lax.axis_index("s") * _NC + lax.axis_index("c")

    @pl.when(wid < _GATHER_WORKERS)
    def _():
        base = wid * _ROWS_PER_WORKER
        pltpu.sync_copy(idx_hbm.at[pl.ds(base, _ROWS_PER_WORKER)], idx_v)
        pltpu.async_copy(x_hbm.at[idx_v], rows_v, sem).wait()
        pltpu.sync_copy(rows_v, out_hbm.at[pl.ds(base, _ROWS_PER_WORKER)])


def kernel(x, cluster_centers):
    x2d = x[0]
    idx = _tc_argmin(x2d, cluster_centers)
    selected = _sc_gather(x2d, idx.reshape(N_CLUSTERS))
    return selected[None]


# speedup vs baseline: 15.9349x; 15.9349x over previous
import functools

import jax
import jax.numpy as jnp
from jax import lax
from jax.experimental import pallas as pl
from jax.experimental.pallas import tpu as pltpu
from jax.experimental.pallas import tpu_sc as plsc

N_POINTS = 16384
N_FEAT = 64
N_CLUSTERS = 128
BLK = 2048
N_BLK = N_POINTS // BLK

_WORKERS = 8
_ROWS_PER_WORKER = N_CLUSTERS // _WORKERS
_PAIRS_PER_WORKER = _ROWS_PER_WORKER // 2


def _argmin_body(x_ref, c_ref, idx2_ref, offs_ref, bv_ref, bi_ref):
    j = pl.program_id(0)
    xb = x_ref[...]
    cb = c_ref[...]
    dot = lax.dot_general(
        xb, cb, (((1,), (1,)), ((), ())), preferred_element_type=jnp.float32
    )
    xn = jnp.sum(xb * xb, axis=1, keepdims=True)
    s = xn - 2.0 * dot
    bmin = jnp.min(s, axis=0, keepdims=True)
    ii = lax.broadcasted_iota(jnp.int32, s.shape, 0) + j * BLK
    bidx = jnp.min(
        jnp.where(s == bmin, ii, jnp.int32(2**30)), axis=0, keepdims=True
    )

    @pl.when(j == 0)
    def _():
        bv_ref[...] = bmin
        bi_ref[...] = bidx

    @pl.when(j > 0)
    def _():
        better = bmin < bv_ref[...]
        bv_ref[...] = jnp.where(better, bmin, bv_ref[...])
        bi_ref[...] = jnp.where(better, bidx, bi_ref[...])

    @pl.when(j == N_BLK - 1)
    def _():
        best = bi_ref[...]
        idx2_ref[...] = best >> 1
        offs_ref[...] = (best & 1) * N_FEAT


def _tc_argmin(x2d, centers):
    return pl.pallas_call(
        _argmin_body,
        grid=(N_BLK,),
        in_specs=[
            pl.BlockSpec((BLK, N_FEAT), lambda j: (j, 0)),
            pl.BlockSpec((N_CLUSTERS, N_FEAT), lambda j: (0, 0)),
        ],
        out_specs=[
            pl.BlockSpec((1, N_CLUSTERS), lambda j: (0, 0)),
            pl.BlockSpec((1, N_CLUSTERS), lambda j: (0, 0)),
        ],
        out_shape=[
            jax.ShapeDtypeStruct((1, N_CLUSTERS), jnp.int32),
            jax.ShapeDtypeStruct((1, N_CLUSTERS), jnp.int32),
        ],
        scratch_shapes=[
            pltpu.VMEM((1, N_CLUSTERS), jnp.float32),
            pltpu.VMEM((1, N_CLUSTERS), jnp.int32),
        ],
    )(x2d, centers)


@functools.lru_cache(maxsize=1)
def _make_sc_gather():
    nc = plsc.get_sparse_core_info().num_cores

    @functools.partial(
        pl.kernel,
        mesh=plsc.VectorSubcoreMesh(core_axis_name="c", subcore_axis_name="s"),
        out_type=jax.ShapeDtypeStruct((N_CLUSTERS // 2, 2 * N_FEAT), jnp.float32),
        scratch_types=[
            pltpu.VMEM((_ROWS_PER_WORKER,), jnp.int32),
            pltpu.VMEM((_ROWS_PER_WORKER,), jnp.int32),
            pltpu.VMEM((_ROWS_PER_WORKER, 2 * N_FEAT), jnp.float32),
            pltpu.VMEM((_PAIRS_PER_WORKER, 2 * N_FEAT), jnp.float32),
            pltpu.SemaphoreType.DMA,
        ],
    )
    def _sc_gather(x2_hbm, idx2_hbm, offs_hbm, out_hbm, idx2_v, offs_v, rows_v, out_v, sem):
        wid = lax.axis_index("s") * nc + lax.axis_index("c")

        @pl.when(wid < _WORKERS)
        def _():
            base = wid * _ROWS_PER_WORKER
            pltpu.sync_copy(idx2_hbm.at[pl.ds(base, _ROWS_PER_WORKER)], idx2_v)
            pltpu.sync_copy(offs_hbm.at[pl.ds(base, _ROWS_PER_WORKER)], offs_v)
            pltpu.async_copy(x2_hbm.at[idx2_v], rows_v, sem).wait()
            offs16 = offs_v[...]
            for q in range(_PAIRS_PER_WORKER):
                for h in range(2):
                    r = 2 * q + h
                    rvec = jnp.full((16,), r, jnp.int32)
                    m = offs16.at[rvec].get(mode="promise_in_bounds").astype(
                        jnp.float32
                    ) * (1.0 / N_FEAT)
                    for c4 in range(N_FEAT // 16):
                        left = rows_v[r, pl.ds(16 * c4, 16)]
                        right = rows_v[r, pl.ds(N_FEAT + 16 * c4, 16)]
                        out_v[q, pl.ds(N_FEAT * h + 16 * c4, 16)] = (
                            left * (1.0 - m) + right * m
                        )
            pltpu.sync_copy(
                out_v, out_hbm.at[pl.ds(wid * _PAIRS_PER_WORKER, _PAIRS_PER_WORKER)]
            )

    return _sc_gather


def kernel(x, cluster_centers):
    x2d = x[0]
    idx2, offs = _tc_argmin(x2d, cluster_centers)
    x_pairs = x2d.reshape(N_POINTS // 2, 2 * N_FEAT)
    out_pairs = _make_sc_gather()(
        x_pairs, idx2.reshape(N_CLUSTERS), offs.reshape(N_CLUSTERS)
    )
    return out_pairs.reshape(1, N_CLUSTERS, N_FEAT)


# speedup vs baseline: 17.9039x; 1.1236x over previous
import functools

import jax
import jax.numpy as jnp
from jax import lax
from jax.experimental import pallas as pl
from jax.experimental.pallas import tpu as pltpu
from jax.experimental.pallas import tpu_sc as plsc

N_POINTS = 16384
N_FEAT = 64
N_CLUSTERS = 128
BLK = 2048
N_BLK = N_POINTS // BLK

_WORKERS = 8
_ROWS_PER_WORKER = N_CLUSTERS // _WORKERS
_PAIRS_PER_WORKER = _ROWS_PER_WORKER // 2


def _argmin_body(x_ref, c_ref, idx_ref, dup_ref, bv_ref, bi_ref):
    j = pl.program_id(0)
    xb = x_ref[0]
    dup_ref[:, 0:N_FEAT] = xb
    dup_ref[:, N_FEAT : 2 * N_FEAT] = xb
    cb = c_ref[...]
    dot = lax.dot_general(
        xb, cb, (((1,), (1,)), ((), ())), preferred_element_type=jnp.float32
    )
    xn = jnp.sum(xb * xb, axis=1, keepdims=True)
    s = xn - 2.0 * dot
    bmin = jnp.min(s, axis=0, keepdims=True)
    ii = lax.broadcasted_iota(jnp.int32, s.shape, 0) + j * BLK
    bidx = jnp.min(
        jnp.where(s == bmin, ii, jnp.int32(2**30)), axis=0, keepdims=True
    )

    @pl.when(j == 0)
    def _():
        bv_ref[...] = bmin
        bi_ref[...] = bidx

    @pl.when(j > 0)
    def _():
        better = bmin < bv_ref[...]
        bv_ref[...] = jnp.where(better, bmin, bv_ref[...])
        bi_ref[...] = jnp.where(better, bidx, bi_ref[...])

    @pl.when(j == N_BLK - 1)
    def _():
        idx_ref[...] = bi_ref[...]


def _tc_argmin(x3d, centers):
    return pl.pallas_call(
        _argmin_body,
        grid=(N_BLK,),
        in_specs=[
            pl.BlockSpec((1, BLK, N_FEAT), lambda j: (0, j, 0)),
            pl.BlockSpec((N_CLUSTERS, N_FEAT), lambda j: (0, 0)),
        ],
        out_specs=[
            pl.BlockSpec((1, N_CLUSTERS), lambda j: (0, 0)),
            pl.BlockSpec((BLK, 2 * N_FEAT), lambda j: (j, 0)),
        ],
        out_shape=[
            jax.ShapeDtypeStruct((1, N_CLUSTERS), jnp.int32),
            jax.ShapeDtypeStruct((N_POINTS, 2 * N_FEAT), jnp.float32),
        ],
        scratch_shapes=[
            pltpu.VMEM((1, N_CLUSTERS), jnp.float32),
            pltpu.VMEM((1, N_CLUSTERS), jnp.int32),
        ],
    )(x3d, centers)


@functools.lru_cache(maxsize=1)
def _make_sc_gather():
    nc = plsc.get_sparse_core_info().num_cores

    @functools.partial(
        pl.kernel,
        mesh=plsc.VectorSubcoreMesh(core_axis_name="c", subcore_axis_name="s"),
        out_type=jax.ShapeDtypeStruct((N_CLUSTERS // 2, 2 * N_FEAT), jnp.float32),
        scratch_types=[
            pltpu.VMEM((_ROWS_PER_WORKER,), jnp.int32),
            pltpu.VMEM((_ROWS_PER_WORKER, 2 * N_FEAT), jnp.float32),
            pltpu.VMEM((_PAIRS_PER_WORKER, 2 * N_FEAT), jnp.float32),
            pltpu.SemaphoreType.DMA,
        ],
    )
    def _sc_gather(dup_hbm, idx_hbm, out_hbm, idx_v, rows_v, out_v, sem):
        wid = lax.axis_index("s") * nc + lax.axis_index("c")

        @pl.when(wid < _WORKERS)
        def _():
            base = wid * _ROWS_PER_WORKER
            pltpu.sync_copy(idx_hbm.at[pl.ds(base, _ROWS_PER_WORKER)], idx_v)
            pltpu.async_copy(dup_hbm.at[idx_v], rows_v, sem).wait()
            for q in range(_PAIRS_PER_WORKER):
                for h in range(2):
                    for c4 in range(N_FEAT // 16):
                        out_v[q, pl.ds(N_FEAT * h + 16 * c4, 16)] = rows_v[
                            2 * q + h, pl.ds(16 * c4, 16)
                        ]
            pltpu.sync_copy(
                out_v, out_hbm.at[pl.ds(wid * _PAIRS_PER_WORKER, _PAIRS_PER_WORKER)]
            )

    return _sc_gather


def kernel(x, cluster_centers):
    idx, x_dup = _tc_argmin(x, cluster_centers)
    out_pairs = _make_sc_gather()(x_dup, idx.reshape(N_CLUSTERS))
    return out_pairs.reshape(1, N_CLUSTERS, N_FEAT)
